# baseline (speedup 1.0000x reference)
import functools

import jax
import jax.numpy as jnp
from jax import lax
from jax.experimental import pallas as pl
from jax.experimental.pallas import tpu as pltpu
from jax.experimental.pallas import tpu_sc as plsc

N = 10000
E = 320000
D = 128

NC = 2
NS = 16
NW = NC * NS

K = 128
CHUNKS = 81
IPH = 3
IC = CHUNKS // IPH
NBUF = 3
EPAD = NW * CHUNKS * K
NPAD = N + 240
ZROWS = NPAD // NS
OROWS = 624

_mesh = plsc.VectorSubcoreMesh(core_axis_name="c", subcore_axis_name="s")


@functools.partial(
    pl.kernel,
    mesh=_mesh,
    compiler_params=pltpu.CompilerParams(use_tc_tiling_on_sc=False),
    out_type=jax.ShapeDtypeStruct((NC, N, D), jnp.float32),
    scratch_types=[
        pltpu.VMEM((IC, K), jnp.int32),
        pltpu.VMEM((IC, K), jnp.int32),
        pltpu.VMEM((K, D // 2), jnp.int32),
        pltpu.VMEM((K, D // 2), jnp.int32),
        pltpu.VMEM((K, D // 2), jnp.int32),
        pltpu.VMEM((K, D), jnp.float32),
        pltpu.VMEM_SHARED((NPAD, D), jnp.float32),
        pltpu.SemaphoreType.DMA,
        pltpu.SemaphoreType.DMA,
        pltpu.SemaphoreType.DMA,
        pltpu.SemaphoreType.DMA,
    ],
)
def _mp_sc(x_hbm, ei_hbm, z_hbm, out_hbm, dst_v, src_v, rows_0, rows_1,
           rows_2, rows_f, acc, sem_0, sem_1, sem_2, sem_i):
    cid = lax.axis_index("c")
    sid = lax.axis_index("s")
    wid = cid * NS + sid

    cp_d = pltpu.async_copy(ei_hbm.at[0, wid, 0], dst_v, sem_i)
    cp_s = pltpu.async_copy(ei_hbm.at[1, wid, 0], src_v, sem_i)

    pltpu.sync_copy(z_hbm, acc.at[pl.ds(sid * ZROWS, ZROWS)])
    cp_d.wait()
    cp_s.wait()
    plsc.subcore_barrier()

    _hi_mask = jnp.full((16,), -65536, jnp.int32)

    def _expand(rows_p):
        def _row(r, carry):
            for c in range(D // 32):
                w = rows_p[r, pl.ds(c * 16, 16)]
                lo = jax.lax.bitcast_convert_type(w << 16, jnp.float32)
                hi = jax.lax.bitcast_convert_type(w & _hi_mask, jnp.float32)
                rows_f[r, pl.ds(c * 16, 16)] = lo
                rows_f[r, pl.ds(D // 2 + c * 16, 16)] = hi
            return carry

        lax.fori_loop(0, K, _row, 0)

    bufs = ((rows_0, sem_0), (rows_1, sem_1), (rows_2, sem_2))

    def _step(c, b, prefetch):
        rows_p, sem = bufs[b]
        pltpu.make_async_copy(x_hbm.at[src_v.at[c]], rows_p, sem).wait()
        _expand(rows_p)
        if prefetch:
            pltpu.async_copy(x_hbm.at[src_v.at[c + NBUF]], rows_p, sem)
        pltpu.sync_copy(rows_f, acc.at[dst_v.at[c]], add=True)

    def _group(g, carry):
        c0 = NBUF * g
        for b in range(NBUF):
            _step(c0 + b, b, True)
        return carry

    for ph in range(IPH):
        if ph > 0:
            cp_d = pltpu.async_copy(ei_hbm.at[0, wid, ph], dst_v, sem_i)
            cp_s = pltpu.async_copy(ei_hbm.at[1, wid, ph], src_v, sem_i)
            cp_d.wait()
            cp_s.wait()
        for b in range(NBUF):
            pltpu.async_copy(x_hbm.at[src_v.at[b]], bufs[b][0], bufs[b][1])
        lax.fori_loop(0, IC // NBUF - 1, _group, 0)
        cl = IC - NBUF
        for b in range(NBUF):
            _step(cl + b, b, False)

    plsc.subcore_barrier()

    ob = sid * OROWS
    pltpu.sync_copy(acc.at[pl.ds(ob, OROWS)],
                    out_hbm.at[cid, pl.ds(ob, OROWS)])

    @pl.when(sid == 0)
    def _tail():
        t0 = NS * OROWS
        pltpu.sync_copy(acc.at[pl.ds(t0, N - t0)],
                        out_hbm.at[cid, pl.ds(t0, N - t0)])


def _combine(parts):
    def _add(p_ref, o_ref):
        o_ref[...] = p_ref[0] + p_ref[1]

    return pl.pallas_call(
        _add,
        grid=(10,),
        in_specs=[pl.BlockSpec((2, N // 10, D), lambda i: (0, i, 0))],
        out_specs=pl.BlockSpec((N // 10, D), lambda i: (i, 0)),
        out_shape=jax.ShapeDtypeStruct((N, D), jnp.float32),
    )(parts)


def kernel(x, edge_index):
    pad = EPAD - E
    dst = jnp.concatenate([edge_index[0], jnp.full((pad,), N, jnp.int32)])
    src = jnp.concatenate([edge_index[1], jnp.zeros((pad,), jnp.int32)])
    ei = jnp.stack([dst, src]).reshape(2, NW, IPH, IC, K)
    z = jnp.zeros((ZROWS, D), jnp.float32)
    xb = x.astype(jnp.bfloat16)
    xp = jax.lax.bitcast_convert_type(
        jnp.stack([xb[:, :D // 2], xb[:, D // 2:]], axis=-1), jnp.int32)
    parts = _mp_sc(xp, ei, z)
    return _combine(parts)


# speedup vs baseline: 1.1813x; 1.1813x over previous
import functools

import jax
import jax.numpy as jnp
from jax import lax
from jax.experimental import pallas as pl
from jax.experimental.pallas import tpu as pltpu
from jax.experimental.pallas import tpu_sc as plsc

N = 10000
E = 320000
D = 128

NC = 2
NS = 16
NW = NC * NS

K = 128
CHUNKS = 80
IPH = 2
IC = CHUNKS // IPH
EPAD = NW * CHUNKS * K
NPAD = N + 240
ZROWS = NPAD // NS
OROWS = 624

_mesh = plsc.VectorSubcoreMesh(core_axis_name="c", subcore_axis_name="s")


@functools.partial(
    pl.kernel,
    mesh=_mesh,
    compiler_params=pltpu.CompilerParams(use_tc_tiling_on_sc=False),
    out_type=jax.ShapeDtypeStruct((NC, N, D), jnp.float32),
    scratch_types=[
        pltpu.VMEM((IC, K), jnp.int32),
        pltpu.VMEM((IC, K), jnp.int32),
        pltpu.VMEM((K, D // 2), jnp.int32),
        pltpu.VMEM((K, D // 2), jnp.int32),
        pltpu.VMEM((K, D), jnp.float32),
        pltpu.VMEM_SHARED((NPAD, D), jnp.float32),
        pltpu.SemaphoreType.DMA,
        pltpu.SemaphoreType.DMA,
        pltpu.SemaphoreType.DMA,
        pltpu.SemaphoreType.DMA,
    ],
)
def _mp_sc(x_hbm, ei_hbm, z_hbm, out_hbm, dst_v, src_v, rows_a, rows_b,
           rows_f, acc, sem_a, sem_b, ssem, sem_i):
    cid = lax.axis_index("c")
    sid = lax.axis_index("s")
    wid = cid * NS + sid

    cp_d = pltpu.async_copy(ei_hbm.at[0, wid, 0], dst_v, sem_i)
    cp_s = pltpu.async_copy(ei_hbm.at[1, wid, 0], src_v, sem_i)

    pltpu.sync_copy(z_hbm, acc.at[pl.ds(sid * ZROWS, ZROWS)])
    cp_d.wait()
    cp_s.wait()
    plsc.subcore_barrier()

    _hi_mask = jnp.full((16,), -65536, jnp.int32)

    def _expand(rows_p):
        def _row(r, carry):
            for c in range(D // 32):
                w = rows_p[r, pl.ds(c * 16, 16)]
                lo = jax.lax.bitcast_convert_type(w << 16, jnp.float32)
                hi = jax.lax.bitcast_convert_type(w & _hi_mask, jnp.float32)
                rows_f[r, pl.ds(c * 16, 16)] = lo
                rows_f[r, pl.ds(D // 2 + c * 16, 16)] = hi
            return carry

        lax.fori_loop(0, K, _row, 0)

    def _wait_scatter(c):
        pltpu.make_async_copy(rows_f, acc.at[dst_v.at[c]], ssem).wait()

    def _group(g, carry):
        c1 = 2 * g + 1
        c2 = c1 + 1
        pltpu.make_async_copy(x_hbm.at[src_v.at[c1]], rows_b, sem_b).wait()
        _wait_scatter(c1 - 1)
        _expand(rows_b)
        pltpu.async_copy(x_hbm.at[src_v.at[c1 + 2]], rows_b, sem_b)
        pltpu.async_copy(rows_f, acc.at[dst_v.at[c1]], ssem, add=True)
        pltpu.make_async_copy(x_hbm.at[src_v.at[c2]], rows_a, sem_a).wait()
        _wait_scatter(c2 - 1)
        _expand(rows_a)
        pltpu.async_copy(x_hbm.at[src_v.at[c2 + 2]], rows_a, sem_a)
        pltpu.async_copy(rows_f, acc.at[dst_v.at[c2]], ssem, add=True)
        return carry

    for ph in range(IPH):
        if ph > 0:
            cp_d = pltpu.async_copy(ei_hbm.at[0, wid, ph], dst_v, sem_i)
            cp_s = pltpu.async_copy(ei_hbm.at[1, wid, ph], src_v, sem_i)
            cp_d.wait()
            cp_s.wait()
        pltpu.async_copy(x_hbm.at[src_v.at[0]], rows_a, sem_a)
        pltpu.async_copy(x_hbm.at[src_v.at[1]], rows_b, sem_b)
        pltpu.make_async_copy(x_hbm.at[src_v.at[0]], rows_a, sem_a).wait()
        _expand(rows_a)
        pltpu.async_copy(x_hbm.at[src_v.at[2]], rows_a, sem_a)
        pltpu.async_copy(rows_f, acc.at[dst_v.at[0]], ssem, add=True)
        lax.fori_loop(0, IC // 2 - 2, _group, 0)
        cp = IC - 3
        pltpu.make_async_copy(x_hbm.at[src_v.at[cp]], rows_b, sem_b).wait()
        _wait_scatter(cp - 1)
        _expand(rows_b)
        pltpu.async_copy(x_hbm.at[src_v.at[cp + 2]], rows_b, sem_b)
        pltpu.async_copy(rows_f, acc.at[dst_v.at[cp]], ssem, add=True)
        pltpu.make_async_copy(x_hbm.at[src_v.at[cp + 1]], rows_a, sem_a).wait()
        _wait_scatter(cp)
        _expand(rows_a)
        pltpu.async_copy(rows_f, acc.at[dst_v.at[cp + 1]], ssem, add=True)
        pltpu.make_async_copy(x_hbm.at[src_v.at[IC - 1]], rows_b, sem_b).wait()
        _wait_scatter(cp + 1)
        _expand(rows_b)
        pltpu.async_copy(rows_f, acc.at[dst_v.at[IC - 1]], ssem, add=True)
        _wait_scatter(IC - 1)

    plsc.subcore_barrier()

    ob = sid * OROWS
    pltpu.sync_copy(acc.at[pl.ds(ob, OROWS)],
                    out_hbm.at[cid, pl.ds(ob, OROWS)])

    @pl.when(sid == 0)
    def _tail():
        t0 = NS * OROWS
        pltpu.sync_copy(acc.at[pl.ds(t0, N - t0)],
                        out_hbm.at[cid, pl.ds(t0, N - t0)])


def _combine(parts):
    def _add(p_ref, o_ref):
        o_ref[...] = p_ref[0] + p_ref[1]

    return pl.pallas_call(
        _add,
        grid=(10,),
        in_specs=[pl.BlockSpec((2, N // 10, D), lambda i: (0, i, 0))],
        out_specs=pl.BlockSpec((N // 10, D), lambda i: (i, 0)),
        out_shape=jax.ShapeDtypeStruct((N, D), jnp.float32),
    )(parts)


def kernel(x, edge_index):
    pad = EPAD - E
    dst = jnp.concatenate([edge_index[0], jnp.full((pad,), N, jnp.int32)])
    src = jnp.concatenate([edge_index[1], jnp.zeros((pad,), jnp.int32)])
    ei = jnp.stack([dst, src]).reshape(2, NW, IPH, IC, K)
    z = jnp.zeros((ZROWS, D), jnp.float32)
    xb = x.astype(jnp.bfloat16)
    xp = jax.lax.bitcast_convert_type(
        jnp.stack([xb[:, :D // 2], xb[:, D // 2:]], axis=-1), jnp.int32)
    parts = _mp_sc(xp, ei, z)
    return _combine(parts)
